# Initial kernel scaffold; baseline (speedup 1.0000x reference)
import jax
import jax.numpy as jnp
from jax.experimental import pallas as pl


def kernel(x, edge_index, D_v_beta, D_e_beta_inv, D_e_alpha, D_v_alpha_inv, W1v, b1v, W1e, b1e, W2v, b2v, W2e, b2e):
    raise NotImplementedError("write your pallas kernel here")


# speedup vs baseline: 7.7367x; 7.7367x over previous
import functools

import jax
import jax.numpy as jnp
from jax import lax
from jax.experimental import pallas as pl
from jax.experimental.pallas import tpu as pltpu
from jax.experimental.pallas import tpu_sc as plsc

N_ROWS = 10000
D = 128
E_TOTAL = 320000
NC, NS = 2, 16
NW = NC * NS
EDGES_PER_TILE = E_TOTAL // NW
CHUNK = 80
NCHUNK = EDGES_PER_TILE // CHUNK
N_PAD = 10240
ROWS_PER_TILE = N_PAD // NS
ZROWS = 128


def _make_propagate():
    mesh = plsc.VectorSubcoreMesh(core_axis_name="c", subcore_axis_name="s")

    def body(table_hbm, gidx_hbm, sidx_hbm, out_hbm, gbuf, sbuf, rows, zbuf,
             acc, sem):
        cid = lax.axis_index("c")
        sid = lax.axis_index("s")
        wid = cid * NS + sid
        base = wid * EDGES_PER_TILE

        def zstep(i, _):
            for j in range(D // 16):
                zbuf[i, pl.ds(j * 16, 16)] = jnp.zeros((16,), jnp.float32)
            return 0
        lax.fori_loop(0, ZROWS, zstep, 0)
        row0 = sid * ROWS_PER_TILE
        for r in range(ROWS_PER_TILE // ZROWS):
            pltpu.sync_copy(zbuf, acc.at[pl.ds(row0 + r * ZROWS, ZROWS)])
        plsc.subcore_barrier()

        def step(i, _):
            eb = base + i * CHUNK
            pltpu.sync_copy(gidx_hbm.at[pl.ds(eb, CHUNK)], gbuf)
            pltpu.sync_copy(sidx_hbm.at[pl.ds(eb, CHUNK)], sbuf)
            pltpu.async_copy(table_hbm.at[gbuf], rows, sem).wait()
            pltpu.sync_copy(rows, acc.at[sbuf], add=True)
            return 0
        lax.fori_loop(0, NCHUNK, step, 0)
        plsc.subcore_barrier()

        pltpu.sync_copy(acc.at[pl.ds(row0, ROWS_PER_TILE)],
                        out_hbm.at[cid, pl.ds(row0, ROWS_PER_TILE)])

    return pl.kernel(
        body,
        out_type=jax.ShapeDtypeStruct((NC, N_PAD, D), jnp.float32),
        mesh=mesh,
        scratch_types=[
            pltpu.VMEM((CHUNK,), jnp.int32),
            pltpu.VMEM((CHUNK,), jnp.int32),
            pltpu.VMEM((CHUNK, D), jnp.float32),
            pltpu.VMEM((ZROWS, D), jnp.float32),
            pltpu.VMEM_SHARED((N_PAD, D), jnp.float32),
            pltpu.SemaphoreType.DMA,
        ],
    )


_propagate = _make_propagate()


_BLK = 2000
_DOT = functools.partial(
    lax.dot_general,
    dimension_numbers=(((1,), (0,)), ((), ())),
    preferred_element_type=jnp.float32,
    precision=lax.Precision.HIGHEST,
)


def _first_body(x_ref, so_ref, w_ref, bias_ref, o_ref):
    y = _DOT(x_ref[...], w_ref[...]) + bias_ref[...]
    o_ref[...] = y * so_ref[...]


def _mid_body(a_ref, b_ref, si_ref, so_ref, w_ref, bias_ref, o_ref):
    t = (a_ref[...] + b_ref[...]) * si_ref[...]
    t = jnp.maximum(t, 0.0)
    y = _DOT(t, w_ref[...]) + bias_ref[...]
    o_ref[...] = y * so_ref[...]


def _last_body(a_ref, b_ref, si_ref, o_ref):
    o_ref[...] = (a_ref[...] + b_ref[...]) * si_ref[...]


_ROWB = pl.BlockSpec((_BLK, D), lambda i: (i, 0))
_COLB = pl.BlockSpec((_BLK, 1), lambda i: (i, 0))
_WB = pl.BlockSpec((D, D), lambda i: (0, 0))
_BB = pl.BlockSpec((1, D), lambda i: (0, 0))
_GRID = (N_ROWS // _BLK,)
_OSHAPE = jax.ShapeDtypeStruct((N_ROWS, D), jnp.float32)

_dense_first = pl.pallas_call(
    _first_body, grid=_GRID, out_shape=_OSHAPE,
    in_specs=[_ROWB, _COLB, _WB, _BB], out_specs=_ROWB)

_dense_mid = pl.pallas_call(
    _mid_body, grid=_GRID, out_shape=_OSHAPE,
    in_specs=[_ROWB, _ROWB, _COLB, _COLB, _WB, _BB], out_specs=_ROWB)

_dense_last = pl.pallas_call(
    _last_body, grid=_GRID, out_shape=_OSHAPE,
    in_specs=[_ROWB, _ROWB, _COLB], out_specs=_ROWB)


def kernel(x, edge_index, D_v_beta, D_e_beta_inv, D_e_alpha, D_v_alpha_inv,
           W1v, b1v, W1e, b1e, W2v, b2v, W2e, b2e):
    src = edge_index[0]
    dst = edge_index[1]
    dvb = D_v_beta.reshape(N_ROWS, 1)
    debi = D_e_beta_inv.reshape(N_ROWS, 1)
    dea = D_e_alpha.reshape(N_ROWS, 1)
    dvai = D_v_alpha_inv.reshape(N_ROWS, 1)
    b1v_ = b1v.reshape(1, D)
    b1e_ = b1e.reshape(1, D)
    b2v_ = b2v.reshape(1, D)
    b2e_ = b2e.reshape(1, D)

    h1 = _dense_first(x, dvb, W1v, b1v_)
    p = _propagate(h1, src, dst)
    e2 = _dense_mid(p[0, :N_ROWS], p[1, :N_ROWS], debi, dea, W1e, b1e_)
    q = _propagate(e2, dst, src)
    h2 = _dense_mid(q[0, :N_ROWS], q[1, :N_ROWS], dvai, dvb, W2v, b2v_)
    p2 = _propagate(h2, src, dst)
    e2b = _dense_mid(p2[0, :N_ROWS], p2[1, :N_ROWS], debi, dea, W2e, b2e_)
    q2 = _propagate(e2b, dst, src)
    return _dense_last(q2[0, :N_ROWS], q2[1, :N_ROWS], dvai)
